# baseline (speedup 1.0000x reference)
import functools

import jax
import jax.numpy as jnp
from jax import lax
from jax.experimental import pallas as pl
from jax.experimental.pallas import tpu as pltpu
from jax.experimental.pallas import tpu_sc as plsc

NC = 2
NS = 16
NW = NC * NS

B = 4096
S = 200
E = 32
HIDDEN = 256
NUM_CLASSES = 64

RPW = B // NW
S0, S1 = 128, S - 128


def _pool_body(ids_hbm, table_hbm, out_hbm, ids_v, buf0, buf1, out_v, sem0, sem1):
    wid = lax.axis_index("s") * NC + lax.axis_index("c")
    base = wid * RPW
    pltpu.sync_copy(ids_hbm.at[pl.ds(base, RPW)], ids_v)

    def issue(r, buf, sem):
        pltpu.async_copy(table_hbm.at[ids_v.at[r, pl.ds(0, S0)]],
                         buf.at[pl.ds(0, S0)], sem)
        pltpu.async_copy(table_hbm.at[ids_v.at[r, pl.ds(S0, S1)]],
                         buf.at[pl.ds(S0, S1)], sem)

    def drain(buf, sem):
        pltpu.make_async_copy(table_hbm.at[pl.ds(0, S)], buf, sem).wait()

    def reduce_store(r, buf):
        def rbody(j, accs):
            a0, a1 = accs
            return a0 + buf[j, 0:16], a1 + buf[j, 16:32]
        z = jnp.zeros((16,), jnp.float32)
        a0, a1 = lax.fori_loop(0, S, rbody, (z, z), unroll=8)
        scale = jnp.float32(1.0 / S)
        out_v[r, 0:16] = a0 * scale
        out_v[r, 16:32] = a1 * scale

    issue(0, buf0, sem0)

    def body(i, carry):
        a = 2 * i
        b = a + 1
        issue(b, buf1, sem1)
        drain(buf0, sem0)
        reduce_store(a, buf0)

        @pl.when(a + 2 < RPW)
        def _():
            issue(a + 2, buf0, sem0)

        drain(buf1, sem1)
        reduce_store(b, buf1)
        return carry

    lax.fori_loop(0, RPW // 2, body, 0)
    pltpu.sync_copy(out_v, out_hbm.at[pl.ds(base, RPW)])


@jax.jit
def _pool(ids, table):
    mesh = plsc.VectorSubcoreMesh(core_axis_name="c", subcore_axis_name="s",
                                  num_cores=NC, num_subcores=NS)
    kfn = pl.kernel(
        _pool_body,
        out_type=jax.ShapeDtypeStruct((B, E), jnp.float32),
        mesh=mesh,
        scratch_types=[
            pltpu.VMEM((RPW, S), jnp.int32),
            pltpu.VMEM((S, E), jnp.float32),
            pltpu.VMEM((S, E), jnp.float32),
            pltpu.VMEM((RPW, E), jnp.float32),
            pltpu.SemaphoreType.DMA,
            pltpu.SemaphoreType.DMA,
        ],
        compiler_params=pltpu.CompilerParams(use_tc_tiling_on_sc=False),
    )
    return kfn(ids, table)


def _pool_w_body(ids_hbm, cols_hbm, table_hbm, out_hbm,
                 ids_v, cols_v, buf0, buf1, out_v, sem0, sem1):
    wid = lax.axis_index("s") * NC + lax.axis_index("c")
    base = wid * (RPW * S)
    pltpu.sync_copy(ids_hbm.at[pl.ds(base, RPW * S)], ids_v)
    pltpu.sync_copy(cols_hbm.at[pl.ds(base, RPW * S)],
                    cols_v.at[pl.ds(0, RPW * S)])

    def issue(r, buf, sem):
        off = r * S
        pltpu.async_copy(table_hbm.at[ids_v.at[pl.ds(off, S0)]],
                         buf.at[pl.ds(0, S0)], sem)
        pltpu.async_copy(table_hbm.at[ids_v.at[pl.ds(off + S0, S1)]],
                         buf.at[pl.ds(S0, S1)], sem)

    def drain(buf, sem):
        pltpu.make_async_copy(table_hbm.at[pl.ds(0, S)], buf, sem).wait()

    def reduce_store(r, buf):
        off = r * S

        def rbody(bi, accs):
            a0, a1 = accs
            jb = bi * 8
            cvec = cols_v[pl.ds(off + jb, 16)]
            for l in range(8):
                c = cvec[l]
                a0 = a0 + buf[jb + l, pl.ds(c, 16)]
                a1 = a1 + buf[jb + l, pl.ds(c + 16, 16)]
            return a0, a1
        z = jnp.zeros((16,), jnp.float32)
        a0, a1 = lax.fori_loop(0, S // 8, rbody, (z, z))
        scale = jnp.float32(1.0 / S)
        out_v[r, 0:16] = a0 * scale
        out_v[r, 16:32] = a1 * scale

    issue(0, buf0, sem0)

    def body(i, carry):
        a = 2 * i
        b = a + 1
        issue(b, buf1, sem1)
        drain(buf0, sem0)
        reduce_store(a, buf0)

        @pl.when(a + 2 < RPW)
        def _():
            issue(a + 2, buf0, sem0)

        drain(buf1, sem1)
        reduce_store(b, buf1)
        return carry

    lax.fori_loop(0, RPW // 2, body, 0)
    pltpu.sync_copy(out_v, out_hbm.at[pl.ds(wid * RPW, RPW)])


@jax.jit
def _pool_w(ids_wide, cols, table_wide):
    mesh = plsc.VectorSubcoreMesh(core_axis_name="c", subcore_axis_name="s",
                                  num_cores=NC, num_subcores=NS)
    kfn = pl.kernel(
        _pool_w_body,
        out_type=jax.ShapeDtypeStruct((B, E), jnp.float32),
        mesh=mesh,
        scratch_types=[
            pltpu.VMEM((RPW * S,), jnp.int32),
            pltpu.VMEM((RPW * S + 16,), jnp.int32),
            pltpu.VMEM((S, 128), jnp.float32),
            pltpu.VMEM((S, 128), jnp.float32),
            pltpu.VMEM((RPW, E), jnp.float32),
            pltpu.SemaphoreType.DMA,
            pltpu.SemaphoreType.DMA,
        ],
    )
    return kfn(ids_wide, cols, table_wide)


N_VOCAB = 1000000
W = 128
N_WIDE = N_VOCAB * E // W
TCH = 4096
TCH_OUT = TCH * E // W
N_TBLK = (N_VOCAB + TCH - 1) // TCH


def _retile_tc_body(y_ref, out_ref):
    yt = jnp.swapaxes(y_ref[...], 0, 1)
    yt3 = yt.reshape(TCH_OUT, 4, E)
    out_ref[...] = jnp.concatenate([yt3[:, h, :] for h in range(4)], axis=1)


@jax.jit
def _retile_tc(tab_t):
    return pl.pallas_call(
        _retile_tc_body,
        grid=(N_TBLK,),
        in_specs=[pl.BlockSpec((E, TCH), lambda i: (0, i))],
        out_specs=pl.BlockSpec((TCH_OUT, W), lambda i: (i, 0)),
        out_shape=jax.ShapeDtypeStruct((N_WIDE, W), jnp.float32),
    )(tab_t)


def _mlp_body(x_ref, w1_ref, b1_ref, w2_ref, b2_ref, out_ref, relu_ref):
    h = jnp.dot(x_ref[...], w1_ref[...], preferred_element_type=jnp.float32)
    h = jnp.maximum(h + b1_ref[...], 0.0)
    relu_ref[...] = h
    out_ref[...] = (jnp.dot(h, w2_ref[...], preferred_element_type=jnp.float32)
                    + b2_ref[...])


@jax.jit
def _mlp(x, w1, b1, w2, b2):
    blk = 512
    grid = B // blk
    return pl.pallas_call(
        _mlp_body,
        grid=(grid,),
        in_specs=[
            pl.BlockSpec((blk, E), lambda i: (i, 0)),
            pl.BlockSpec((E, HIDDEN), lambda i: (0, 0)),
            pl.BlockSpec((1, HIDDEN), lambda i: (0, 0)),
            pl.BlockSpec((HIDDEN, NUM_CLASSES), lambda i: (0, 0)),
            pl.BlockSpec((1, NUM_CLASSES), lambda i: (0, 0)),
        ],
        out_specs=[
            pl.BlockSpec((blk, NUM_CLASSES), lambda i: (i, 0)),
            pl.BlockSpec((blk, HIDDEN), lambda i: (i, 0)),
        ],
        out_shape=[
            jax.ShapeDtypeStruct((B, NUM_CLASSES), jnp.float32),
            jax.ShapeDtypeStruct((B, HIDDEN), jnp.float32),
        ],
    )(x, w1, b1, w2, b2)


def kernel(input_ids, label, attention_mask, emb_table, W1, b1, W2, b2):
    ids = input_ids.astype(jnp.int32).reshape(-1)
    ids_wide = ids // (W // E)
    cols = (ids % (W // E)) * E
    table_wide = _retile_tc(emb_table.T)
    pooled = _pool_w(ids_wide, cols, table_wide)
    out, out_relu = _mlp(pooled, W1, b1.reshape(1, HIDDEN),
                         W2, b2.reshape(1, NUM_CLASSES))
    return out, out_relu


# speedup vs baseline: 1.0621x; 1.0621x over previous
import functools

import jax
import jax.numpy as jnp
from jax import lax
from jax.experimental import pallas as pl
from jax.experimental.pallas import tpu as pltpu
from jax.experimental.pallas import tpu_sc as plsc

NC = 2
NS = 16
NW = NC * NS

B = 4096
S = 200
E = 32
HIDDEN = 256
NUM_CLASSES = 64

RPW = B // NW
S0, S1 = 128, S - 128


def _pool_body(ids_hbm, table_hbm, out_hbm, ids_v, buf0, buf1, out_v, sem0, sem1):
    wid = lax.axis_index("s") * NC + lax.axis_index("c")
    base = wid * RPW
    pltpu.sync_copy(ids_hbm.at[pl.ds(base, RPW)], ids_v)

    def issue(r, buf, sem):
        pltpu.async_copy(table_hbm.at[ids_v.at[r, pl.ds(0, S0)]],
                         buf.at[pl.ds(0, S0)], sem)
        pltpu.async_copy(table_hbm.at[ids_v.at[r, pl.ds(S0, S1)]],
                         buf.at[pl.ds(S0, S1)], sem)

    def drain(buf, sem):
        pltpu.make_async_copy(table_hbm.at[pl.ds(0, S)], buf, sem).wait()

    def reduce_store(r, buf):
        def rbody(j, accs):
            a0, a1 = accs
            return a0 + buf[j, 0:16], a1 + buf[j, 16:32]
        z = jnp.zeros((16,), jnp.float32)
        a0, a1 = lax.fori_loop(0, S, rbody, (z, z), unroll=8)
        scale = jnp.float32(1.0 / S)
        out_v[r, 0:16] = a0 * scale
        out_v[r, 16:32] = a1 * scale

    issue(0, buf0, sem0)

    def body(i, carry):
        a = 2 * i
        b = a + 1
        issue(b, buf1, sem1)
        drain(buf0, sem0)
        reduce_store(a, buf0)

        @pl.when(a + 2 < RPW)
        def _():
            issue(a + 2, buf0, sem0)

        drain(buf1, sem1)
        reduce_store(b, buf1)
        return carry

    lax.fori_loop(0, RPW // 2, body, 0)
    pltpu.sync_copy(out_v, out_hbm.at[pl.ds(base, RPW)])


@jax.jit
def _pool(ids, table):
    mesh = plsc.VectorSubcoreMesh(core_axis_name="c", subcore_axis_name="s",
                                  num_cores=NC, num_subcores=NS)
    kfn = pl.kernel(
        _pool_body,
        out_type=jax.ShapeDtypeStruct((B, E), jnp.float32),
        mesh=mesh,
        scratch_types=[
            pltpu.VMEM((RPW, S), jnp.int32),
            pltpu.VMEM((S, E), jnp.float32),
            pltpu.VMEM((S, E), jnp.float32),
            pltpu.VMEM((RPW, E), jnp.float32),
            pltpu.SemaphoreType.DMA,
            pltpu.SemaphoreType.DMA,
        ],
        compiler_params=pltpu.CompilerParams(use_tc_tiling_on_sc=False),
    )
    return kfn(ids, table)


def _mlp_body(x_ref, w1_ref, b1_ref, w2_ref, b2_ref, out_ref, relu_ref):
    h = jnp.dot(x_ref[...], w1_ref[...], preferred_element_type=jnp.float32)
    h = jnp.maximum(h + b1_ref[...], 0.0)
    relu_ref[...] = h
    out_ref[...] = (jnp.dot(h, w2_ref[...], preferred_element_type=jnp.float32)
                    + b2_ref[...])


@jax.jit
def _mlp(x, w1, b1, w2, b2):
    blk = 512
    grid = B // blk
    return pl.pallas_call(
        _mlp_body,
        grid=(grid,),
        in_specs=[
            pl.BlockSpec((blk, E), lambda i: (i, 0)),
            pl.BlockSpec((E, HIDDEN), lambda i: (0, 0)),
            pl.BlockSpec((1, HIDDEN), lambda i: (0, 0)),
            pl.BlockSpec((HIDDEN, NUM_CLASSES), lambda i: (0, 0)),
            pl.BlockSpec((1, NUM_CLASSES), lambda i: (0, 0)),
        ],
        out_specs=[
            pl.BlockSpec((blk, NUM_CLASSES), lambda i: (i, 0)),
            pl.BlockSpec((blk, HIDDEN), lambda i: (i, 0)),
        ],
        out_shape=[
            jax.ShapeDtypeStruct((B, NUM_CLASSES), jnp.float32),
            jax.ShapeDtypeStruct((B, HIDDEN), jnp.float32),
        ],
    )(x, w1, b1, w2, b2)


def kernel(input_ids, label, attention_mask, emb_table, W1, b1, W2, b2):
    ids = input_ids.astype(jnp.int32)
    flat = lax.optimization_barrier(emb_table.reshape(-1))
    pooled = _pool(ids, flat.reshape(emb_table.shape))
    out, out_relu = _mlp(pooled, W1, b1.reshape(1, HIDDEN),
                         W2, b2.reshape(1, NUM_CLASSES))
    return out, out_relu
